# baseline (speedup 1.0000x reference)
import functools

import jax
import jax.numpy as jnp
from jax import lax
from jax.experimental import pallas as pl
from jax.experimental.pallas import tpu as pltpu
from jax.experimental.pallas import tpu_sc as plsc

_NC = 2
_NS = 16
_NW = _NC * _NS
_L = 16


@functools.lru_cache(maxsize=None)
def _make_gather(B0, H, V2, D):
    rows_per_w = B0 // _NW
    C = H
    HR = H // 2
    n_chunks = rows_per_w
    n_pairs = n_chunks // 2
    n_grp = (C + _L - 1) // _L
    out_rows = B0 * H // 2
    rows_per_w_out = rows_per_w * HR
    mesh = plsc.VectorSubcoreMesh(
        core_axis_name="c", subcore_axis_name="s",
        num_cores=_NC, num_subcores=_NS,
    )

    @functools.partial(
        pl.kernel,
        out_type=jax.ShapeDtypeStruct((out_rows, 2 * D), jnp.float32),
        mesh=mesh,
        scratch_types=[
            [pltpu.VMEM((C,), jnp.int32)] * 2,
            [pltpu.VMEM((C,), jnp.int32)] * 2,
            [pltpu.VMEM((C,), jnp.int32)] * 2,
            [pltpu.VMEM((C, 2 * D), jnp.float32)] * 2,
            [pltpu.VMEM((2 * HR, 2 * D), jnp.float32)] * 2,
            [pltpu.SemaphoreType.DMA] * 2,
            [pltpu.SemaphoreType.DMA] * 2,
        ],
        compiler_params=pltpu.CompilerParams(use_tc_tiling_on_sc=True,
                                             needs_layout_passes=False),
    )
    def gather_kernel(x_hbm, t2_hbm, out_hbm, idx_v, idx2_v, par_v, g_v,
                      st_v, g_sem, st_sem):
        wid = lax.axis_index("s") * _NC + lax.axis_index("c")
        ibase = wid * rows_per_w * H
        obase = wid * rows_per_w_out

        def load_prep(i, b):
            pltpu.sync_copy(x_hbm.at[pl.ds(ibase + i * C, C)], idx_v[b])

            def prep(j, carry):
                v = idx_v[b][pl.ds(j * _L, _L)]
                idx2_v[b][pl.ds(j * _L, _L)] = lax.shift_right_logical(v, 1)
                par_v[b][pl.ds(j * _L, _L)] = (v & 1) * D
                return carry

            lax.fori_loop(0, C // _L, prep, 0)
            if C % _L:
                v = idx_v[b][pl.ds(C - _L, _L)]
                idx2_v[b][pl.ds(C - _L, _L)] = lax.shift_right_logical(v, 1)
                par_v[b][pl.ds(C - _L, _L)] = (v & 1) * D

        def fire_gather(i, b):
            load_prep(i, b)
            pltpu.async_copy(t2_hbm.at[idx2_v[b]], g_v[b], g_sem[b])

        def reloc(b, B, rowofs):
            pltpu.make_async_copy(t2_hbm.at[idx2_v[b]], g_v[b],
                                  g_sem[b]).wait()
            zeros = jnp.zeros((_L,), jnp.int32)

            def grp(j, carry):
                rows = j * _L + lax.iota(jnp.int32, _L)
                msk = rows < C
                rows = jnp.where(msk, rows, 0)
                par = plsc.load_gather(par_v[b], [rows])
                dst_r = rowofs + lax.shift_right_logical(rows, 1)
                half = (rows & 1) * D
                for c in range(0, D, _L):
                    cc = c + lax.iota(jnp.int32, _L)
                    vals = plsc.load_gather(g_v[b], [rows, par + c],
                                            mask=msk)
                    plsc.store_scatter(st_v[B], [dst_r, half + cc],
                                       vals, mask=msk)
                return carry

            lax.fori_loop(0, n_grp, grp, 0)

        def fire_store(p, B):
            pltpu.async_copy(st_v[B],
                             out_hbm.at[pl.ds(obase + p * 2 * HR, 2 * HR)],
                             st_sem[B])

        def wait_store(B):
            pltpu.make_async_copy(st_v[B], out_hbm.at[pl.ds(0, 2 * HR)],
                                  st_sem[B]).wait()

        fire_gather(0, 0)
        fire_gather(1, 1)
        reloc(0, 0, 0)
        fire_gather(2, 0)
        reloc(1, 0, HR)
        fire_store(0, 0)
        fire_gather(3, 1)
        reloc(0, 1, 0)
        fire_gather(4, 0)
        reloc(1, 1, HR)
        fire_store(1, 1)
        fire_gather(5, 1)

        def body(p, carry):
            B = lax.rem(p, 2)

            @pl.when(B == 0)
            def _():
                wait_store(0)
                reloc(0, 0, 0)
                fire_gather(2 * p + 2, 0)
                reloc(1, 0, HR)
                fire_store(p, 0)
                fire_gather(2 * p + 3, 1)

            @pl.when(B == 1)
            def _():
                wait_store(1)
                reloc(0, 1, 0)
                fire_gather(2 * p + 2, 0)
                reloc(1, 1, HR)
                fire_store(p, 1)
                fire_gather(2 * p + 3, 1)
            return carry

        lax.fori_loop(2, n_pairs - 1, body, 0)

        pB = (n_pairs - 1) % 2
        wait_store(pB)
        reloc(0, pB, 0)
        reloc(1, pB, HR)
        fire_store(n_pairs - 1, pB)
        wait_store(pB)
        wait_store(1 - pB)

    return gather_kernel


def kernel(x, table):
    B0, H = x.shape
    V, D = table.shape
    xf = x.reshape(-1).astype(jnp.int32)
    t2 = table.reshape(V // 2, 2 * D)
    out = _make_gather(B0, H, V // 2, D)(xf, t2)
    return out.reshape(B0, H, D)
